# baseline (speedup 1.0000x reference)
import numpy as np
import jax
import jax.numpy as jnp
from jax.experimental import pallas as pl
from jax.experimental.pallas import tpu as pltpu

_EPS2 = 1e-30
_G = 8


def _body(x_ref, h0_ref, h1_ref, h2_ref, h3_ref, mem_ref, wf_ref, bias_ref,
          eye4_ref, s8_ref, ones_ref, out_ref, *, tseq, rows, w_refs):
    grp = _G * tseq
    n_grp = rows // grp
    hid = h0_ref.shape[-1]
    bias = bias_ref[...]

    x = x_ref[...].reshape(rows, x_ref.shape[-1])
    e = jnp.dot(x, wf_ref[...], preferred_element_type=jnp.float32)
    p = jnp.exp(e)
    mems = jnp.dot(p, mem_ref[...], preferred_element_type=jnp.float32)
    n2m = jnp.dot(mems * mems, ones_ref[...],
                  preferred_element_type=jnp.float32)

    att_parts = []
    for h_ref, w_ref in zip((h0_ref, h1_ref, h2_ref, h3_ref), w_refs):
        qkv = jnp.dot(h_ref[...].reshape(rows, hid), w_ref[...],
                      preferred_element_type=jnp.float32)
        qkv3 = qkv.reshape(n_grp, grp, 3 * 32)
        en = bias + jax.lax.dot_general(
            qkv3[:, :, 0:32], qkv3[:, :, 32:64],
            (((2,), (2,)), ((0,), (0,))),
            preferred_element_type=jnp.float32)
        en = en - jnp.max(en, axis=-1, keepdims=True)
        pr = jnp.exp(en)
        att_parts.append(jax.lax.dot_general(
            pr, qkv3[:, :, 64:96], (((2,), (1,)), ((0,), (0,))),
            preferred_element_type=jnp.float32))
    att = jnp.concatenate(att_parts, axis=2).reshape(rows, 128)
    m4 = jnp.dot(mems, eye4_ref[...],
                 preferred_element_type=jnp.float32)
    packed = jnp.concatenate([att * att, att * m4], axis=1)
    prods = jnp.dot(packed, s8_ref[...],
                    preferred_element_type=jnp.float32)
    scale = jax.lax.rsqrt(jnp.maximum(prods[:, 0:4] * n2m, _EPS2))
    out_ref[...] = prods[:, 4:8] * scale


def kernel(input, hidden_0, hidden_1, hidden_2, hidden_3, memory, input_query,
           hid_query_0, hid_query_1, hid_query_2, hid_query_3,
           key_0, key_1, key_2, key_3,
           value_0, value_1, value_2, value_3):
    B, N, T, IN_DIM = input.shape
    HID = hidden_0.shape[-1]
    MH = memory.shape[1]
    BN = B * N
    total = BN * T

    tile_p = 64
    rows = tile_p * T
    steps = BN // tile_p
    grp = _G * T

    x = input.reshape(BN, T, IN_DIM)
    hs = [h.reshape(BN, T, HID)
          for h in (hidden_0, hidden_1, hidden_2, hidden_3)]
    wf = jnp.dot(input_query, memory.T)
    ws = [jnp.concatenate([hq, kk, vv], axis=1)
          for hq, kk, vv in ((hid_query_0, key_0, value_0),
                             (hid_query_1, key_1, value_1),
                             (hid_query_2, key_2, value_2),
                             (hid_query_3, key_3, value_3))]

    rr = np.arange(grp) // T
    bias = jnp.asarray(
        np.where(rr[:, None] == rr[None, :], 0.0, -1e30), jnp.float32)
    eye4 = jnp.asarray(np.tile(np.eye(MH, dtype=np.float32), (1, 4)))
    s8 = np.zeros((8 * MH, 8), np.float32)
    for j in range(8):
        s8[j * MH:(j + 1) * MH, j] = 1.0
    s8 = jnp.asarray(s8)
    ones = jnp.ones((MH, 1), jnp.float32)

    row_spec = lambda width: pl.BlockSpec((tile_p, T, width),
                                          lambda i: (i, 0, 0))
    full_spec = lambda a: pl.BlockSpec(a.shape, lambda i: (0,) * a.ndim)

    def body_fn(x_ref, h0, h1, h2, h3, mem_ref, wf_ref, bias_ref,
                eye4_ref, s8_ref, ones_ref, w0, w1, w2, w3, out_ref):
        _body(x_ref, h0, h1, h2, h3, mem_ref, wf_ref, bias_ref,
              eye4_ref, s8_ref, ones_ref, out_ref,
              tseq=T, rows=rows, w_refs=(w0, w1, w2, w3))

    scores = pl.pallas_call(
        body_fn,
        grid=(steps,),
        in_specs=[row_spec(IN_DIM)] + [row_spec(HID)] * 4
                 + [full_spec(a) for a in (memory, wf, bias, eye4, s8, ones)]
                 + [full_spec(w) for w in ws],
        out_specs=pl.BlockSpec((rows, 4), lambda i: (i, 0)),
        out_shape=jax.ShapeDtypeStruct((total, 4), jnp.float32),
        compiler_params=pltpu.CompilerParams(
            dimension_semantics=("parallel",)),
    )(x, *hs, memory, wf, bias, eye4, s8, ones, *ws)

    return scores.reshape(B, N, T, 1, 4)


# speedup vs baseline: 1.5408x; 1.5408x over previous
import numpy as np
import jax
import jax.numpy as jnp
from jax.experimental import pallas as pl
from jax.experimental.pallas import tpu as pltpu

_EPS2 = 1e-30
_G = 8


def _body(x_ref, h0_ref, h1_ref, h2_ref, h3_ref, mem_ref, wf_ref, bias_ref,
          eye4_ref, s8_ref, ones_ref, out_ref, *, tseq, rows, w_refs):
    grp = _G * tseq
    n_grp = rows // grp
    hid = h0_ref.shape[-1]
    bias = bias_ref[...]

    x = x_ref[...].reshape(rows, x_ref.shape[-1])
    e = jnp.dot(x, wf_ref[...], preferred_element_type=jnp.float32)
    p = jnp.exp(e)
    mems = jnp.dot(p, mem_ref[...], preferred_element_type=jnp.float32)
    n2m = jnp.dot(mems * mems, ones_ref[...],
                  preferred_element_type=jnp.float32)

    att_parts = []
    for h_ref, w_ref in zip((h0_ref, h1_ref, h2_ref, h3_ref), w_refs):
        qkv = jnp.dot(h_ref[...].reshape(rows, hid), w_ref[...],
                      preferred_element_type=jnp.float32)
        qkv3 = qkv.reshape(n_grp, grp, 3 * 32)
        en = bias + jax.lax.dot_general(
            qkv3[:, :, 0:32], qkv3[:, :, 32:64],
            (((2,), (2,)), ((0,), (0,))),
            preferred_element_type=jnp.float32)
        en = en - jnp.max(en, axis=-1, keepdims=True)
        pr = jnp.exp(en)
        att_parts.append(jax.lax.dot_general(
            pr, qkv3[:, :, 64:96], (((2,), (1,)), ((0,), (0,))),
            preferred_element_type=jnp.float32))
    att = jnp.concatenate(att_parts, axis=2).reshape(rows, 128)
    m4 = jnp.dot(mems, eye4_ref[...],
                 preferred_element_type=jnp.float32)
    packed = jnp.concatenate([att * att, att * m4], axis=1)
    prods = jnp.dot(packed, s8_ref[...],
                    preferred_element_type=jnp.float32)
    scale = jax.lax.rsqrt(jnp.maximum(prods[:, 0:4] * n2m, _EPS2))
    out_ref[...] = prods[:, 4:8] * scale


def kernel(input, hidden_0, hidden_1, hidden_2, hidden_3, memory, input_query,
           hid_query_0, hid_query_1, hid_query_2, hid_query_3,
           key_0, key_1, key_2, key_3,
           value_0, value_1, value_2, value_3):
    B, N, T, IN_DIM = input.shape
    HID = hidden_0.shape[-1]
    MH = memory.shape[1]
    BN = B * N
    total = BN * T

    tile_p = 104
    rows = tile_p * T
    steps = BN // tile_p
    grp = _G * T

    x = input.reshape(BN, T, IN_DIM)
    hs = [h.reshape(BN, T, HID)
          for h in (hidden_0, hidden_1, hidden_2, hidden_3)]
    wf = jnp.dot(input_query, memory.T)
    ws = [jnp.concatenate([hq, kk, vv], axis=1)
          for hq, kk, vv in ((hid_query_0, key_0, value_0),
                             (hid_query_1, key_1, value_1),
                             (hid_query_2, key_2, value_2),
                             (hid_query_3, key_3, value_3))]

    rr = np.arange(grp) // T
    bias = jnp.asarray(
        np.where(rr[:, None] == rr[None, :], 0.0, -1e30), jnp.float32)
    eye4 = jnp.asarray(np.tile(np.eye(MH, dtype=np.float32), (1, 4)))
    s8 = np.zeros((8 * MH, 8), np.float32)
    for j in range(8):
        s8[j * MH:(j + 1) * MH, j] = 1.0
    s8 = jnp.asarray(s8)
    ones = jnp.ones((MH, 1), jnp.float32)

    row_spec = lambda width: pl.BlockSpec((tile_p, T, width),
                                          lambda i: (i, 0, 0))
    full_spec = lambda a: pl.BlockSpec(a.shape, lambda i: (0,) * a.ndim)

    def body_fn(x_ref, h0, h1, h2, h3, mem_ref, wf_ref, bias_ref,
                eye4_ref, s8_ref, ones_ref, w0, w1, w2, w3, out_ref):
        _body(x_ref, h0, h1, h2, h3, mem_ref, wf_ref, bias_ref,
              eye4_ref, s8_ref, ones_ref, out_ref,
              tseq=T, rows=rows, w_refs=(w0, w1, w2, w3))

    scores = pl.pallas_call(
        body_fn,
        grid=(steps,),
        in_specs=[row_spec(IN_DIM)] + [row_spec(HID)] * 4
                 + [full_spec(a) for a in (memory, wf, bias, eye4, s8, ones)]
                 + [full_spec(w) for w in ws],
        out_specs=pl.BlockSpec((rows, 4), lambda i: (i, 0)),
        out_shape=jax.ShapeDtypeStruct((total, 4), jnp.float32),
        compiler_params=pltpu.CompilerParams(
            dimension_semantics=("parallel",)),
    )(x, *hs, memory, wf, bias, eye4, s8, ones, *ws)

    return scores.reshape(B, N, T, 1, 4)
